# baseline (speedup 1.0000x reference)
import jax
import jax.numpy as jnp
from jax import lax
from jax.experimental import pallas as pl
from jax.experimental.pallas import tpu as pltpu
from jax.experimental.pallas import tpu_sc as plsc

BN = 64
NB = 16
N_TILES = 4
N_TOKENS = 1025
D = 1280
ZERO_ROW = 16


def _sc_gather_body(h_hbm, w_hbm, table_hbm, out_hbm, hw_v, idx_v, rows_v, sem):
    cid = lax.axis_index("c")
    sid = lax.axis_index("s")
    wid = sid * 2 + cid

    @pl.when(wid == 0)
    def _():
        pltpu.sync_copy(h_hbm, hw_v.at[pl.ds(0, NB)])
        pltpu.sync_copy(w_hbm, hw_v.at[pl.ds(NB, NB)])
        h = hw_v[pl.ds(0, NB)]
        w = hw_v[pl.ds(NB, NB)]
        n = jnp.where(w < 1, jnp.zeros_like(h), jnp.where(w == 1, h, h + h))
        for t in range(N_TILES):
            e1 = t * N_TILES
            e2 = (t // 2) * N_TILES + (t % 2)
            e = jnp.where(
                t < n,
                jnp.where(w >= 2, jnp.full((NB,), e2, jnp.int32),
                          jnp.full((NB,), e1, jnp.int32)),
                jnp.full((NB,), ZERO_ROW, jnp.int32),
            )
            idx_v[pl.ds(t * NB, NB)] = e
        pltpu.async_copy(table_hbm.at[idx_v], rows_v, sem).wait()
        pltpu.sync_copy(rows_v, out_hbm)


def _sc_gather(h_arr, w_arr, table):
    mesh = plsc.VectorSubcoreMesh(core_axis_name="c", subcore_axis_name="s")
    f = pl.kernel(
        _sc_gather_body,
        out_type=jax.ShapeDtypeStruct((BN, D), jnp.float32),
        mesh=mesh,
        scratch_types=[
            pltpu.VMEM((2 * NB,), jnp.int32),
            pltpu.VMEM((BN,), jnp.int32),
            pltpu.VMEM((BN, D), jnp.float32),
            pltpu.SemaphoreType.DMA,
        ],
    )
    return f(h_arr, w_arr, table)


def _tc_add_body(gate_ref, x_ref, add_ref, o_ref):
    g = jnp.tanh(gate_ref[0])
    o_ref[...] = x_ref[...] + g * add_ref[...]


def _tc_add(gate, xr, addend):
    return pl.pallas_call(
        _tc_add_body,
        grid=(BN,),
        in_specs=[
            pl.BlockSpec(memory_space=pltpu.SMEM),
            pl.BlockSpec((1, N_TOKENS, D), lambda i: (i, 0, 0)),
            pl.BlockSpec((1, 1, D), lambda i: (NB * (i % N_TILES) + i // N_TILES, 0, 0)),
        ],
        out_specs=pl.BlockSpec((1, N_TOKENS, D), lambda i: (i, 0, 0)),
        out_shape=jax.ShapeDtypeStruct((BN, N_TOKENS, D), jnp.float32),
    )(gate, xr, addend.reshape(BN, 1, D))


def kernel(x, aspect_ratio, embedding, gate):
    bsz, n_tiles, n_tokens, d = x.shape
    ar = aspect_ratio.astype(jnp.int32)
    table = jnp.concatenate(
        [embedding.reshape(16, d), jnp.zeros((8, d), jnp.float32)], axis=0
    )
    addend = _sc_gather(ar[:, 0], ar[:, 1], table)
    out = _tc_add(gate, x.reshape(BN, n_tokens, d), addend)
    return out.reshape(bsz, n_tiles, n_tokens, d)


# speedup vs baseline: 3.2547x; 3.2547x over previous
import jax
import jax.numpy as jnp
from jax import lax
from jax.experimental import pallas as pl
from jax.experimental.pallas import tpu as pltpu
from jax.experimental.pallas import tpu_sc as plsc

BN = 64
NB = 16
N_TILES = 4
N_TOKENS = 1025
D = 1280
ZERO_ROW = 16


def _sc_gather_body(h_hbm, w_hbm, table_hbm, out_hbm, hw_v, idx_v, rows_v, sem):
    cid = lax.axis_index("c")
    sid = lax.axis_index("s")
    wid = sid * 2 + cid

    @pl.when(wid == 0)
    def _():
        pltpu.sync_copy(h_hbm, hw_v.at[pl.ds(0, NB)])
        pltpu.sync_copy(w_hbm, hw_v.at[pl.ds(NB, NB)])
        h = hw_v[pl.ds(0, NB)]
        w = hw_v[pl.ds(NB, NB)]
        n = jnp.where(w < 1, jnp.zeros_like(h), jnp.where(w == 1, h, h + h))
        for t in range(N_TILES):
            e1 = t * N_TILES
            e2 = (t // 2) * N_TILES + (t % 2)
            e = jnp.where(
                t < n,
                jnp.where(w >= 2, jnp.full((NB,), e2, jnp.int32),
                          jnp.full((NB,), e1, jnp.int32)),
                jnp.full((NB,), ZERO_ROW, jnp.int32),
            )
            idx_v[pl.ds(t * NB, NB)] = e
        pltpu.async_copy(table_hbm.at[idx_v], rows_v, sem).wait()
        pltpu.sync_copy(rows_v, out_hbm)


def _sc_gather(h_arr, w_arr, table):
    mesh = plsc.VectorSubcoreMesh(core_axis_name="c", subcore_axis_name="s")
    f = pl.kernel(
        _sc_gather_body,
        out_type=jax.ShapeDtypeStruct((BN, D), jnp.float32),
        mesh=mesh,
        scratch_types=[
            pltpu.VMEM((2 * NB,), jnp.int32),
            pltpu.VMEM((BN,), jnp.int32),
            pltpu.VMEM((BN, D), jnp.float32),
            pltpu.SemaphoreType.DMA,
        ],
    )
    return f(h_arr, w_arr, table)


def _tc_add_body(gate_ref, x_ref, add_ref, o_ref):
    g = jnp.tanh(gate_ref[0])
    o_ref[...] = x_ref[...] + g * add_ref[...]


def _tc_add(gate, x, addend):
    return pl.pallas_call(
        _tc_add_body,
        grid=(NB, N_TILES),
        in_specs=[
            pl.BlockSpec(memory_space=pltpu.SMEM),
            pl.BlockSpec((1, 1, N_TOKENS, D), lambda b, t: (b, t, 0, 0)),
            pl.BlockSpec((1, 1, 1, D), lambda b, t: (t, b, 0, 0)),
        ],
        out_specs=pl.BlockSpec((1, 1, N_TOKENS, D), lambda b, t: (b, t, 0, 0)),
        out_shape=jax.ShapeDtypeStruct((NB, N_TILES, N_TOKENS, D), jnp.float32),
    )(gate, x, addend.reshape(N_TILES, NB, 1, D))


def kernel(x, aspect_ratio, embedding, gate):
    bsz, n_tiles, n_tokens, d = x.shape
    ar = aspect_ratio.astype(jnp.int32)
    table = jnp.concatenate(
        [embedding.reshape(16, d), jnp.zeros((8, d), jnp.float32)], axis=0
    )
    addend = _sc_gather(ar[:, 0], ar[:, 1], table)
    return _tc_add(gate, x, addend)
